# baseline (speedup 1.0000x reference)
from functools import partial

import jax
import jax.numpy as jnp
from jax.experimental import pallas as pl
from jax.experimental.pallas import tpu as pltpu

_BLOCK_ROWS = 6272
_SUB_STEPS = 4


def _ln_sub_kernel(x_ref, w_ref, b_ref, o_ref, *, inv_c, eps, sub_rows):
    j = pl.program_id(1)
    x = x_ref[pl.ds(j * sub_rows, sub_rows), :]
    s1 = jnp.sum(x, axis=-1, keepdims=True)
    s2 = jnp.sum(x * x, axis=-1, keepdims=True)
    mean = s1 * inv_c
    var = s2 * inv_c - mean * mean
    rstd = jax.lax.rsqrt(jnp.maximum(var, 0.0) + eps)
    o_ref[...] = (x - mean) * rstd * w_ref[...] + b_ref[...]


def _ln_plain_kernel(x_ref, w_ref, b_ref, o_ref, *, inv_c, eps):
    x = x_ref[...]
    s1 = jnp.sum(x, axis=-1, keepdims=True)
    s2 = jnp.sum(x * x, axis=-1, keepdims=True)
    mean = s1 * inv_c
    var = s2 * inv_c - mean * mean
    rstd = jax.lax.rsqrt(jnp.maximum(var, 0.0) + eps)
    o_ref[...] = (x - mean) * rstd * w_ref[...] + b_ref[...]


def kernel(x, weight, bias):
    eps = 1e-6
    c = x.shape[-1]
    lead = x.shape[:-1]
    rows = 1
    for d in lead:
        rows *= d
    x2d = x.reshape(rows, c)
    w2d = weight.reshape(1, c).astype(jnp.float32)
    b2d = bias.reshape(1, c).astype(jnp.float32)
    cparams = pltpu.CompilerParams(
        dimension_semantics=("parallel", "arbitrary"),
        vmem_limit_bytes=64 * 1024 * 1024,
    )

    bm = _BLOCK_ROWS
    sub = bm // _SUB_STEPS
    if rows % bm == 0:
        nblk = rows // bm
        out = pl.pallas_call(
            partial(_ln_sub_kernel, inv_c=1.0 / c, eps=eps, sub_rows=sub),
            out_shape=jax.ShapeDtypeStruct((rows, c), x.dtype),
            grid=(nblk, _SUB_STEPS),
            in_specs=[
                pl.BlockSpec((bm, c), lambda i, j: (i, 0)),
                pl.BlockSpec((1, c), lambda i, j: (0, 0)),
                pl.BlockSpec((1, c), lambda i, j: (0, 0)),
            ],
            out_specs=pl.BlockSpec((sub, c), lambda i, j: (i * _SUB_STEPS + j, 0)),
            compiler_params=cparams,
        )(x2d, w2d, b2d)
        return out.reshape(*lead, c)

    tm = max(8, min(rows, 2048))
    out = pl.pallas_call(
        partial(_ln_plain_kernel, inv_c=1.0 / c, eps=eps),
        out_shape=jax.ShapeDtypeStruct((rows, c), x.dtype),
        grid=(pl.cdiv(rows, tm),),
        in_specs=[
            pl.BlockSpec((tm, c), lambda i: (i, 0)),
            pl.BlockSpec((1, c), lambda i: (0, 0)),
            pl.BlockSpec((1, c), lambda i: (0, 0)),
        ],
        out_specs=pl.BlockSpec((tm, c), lambda i: (i, 0)),
        compiler_params=pltpu.CompilerParams(
            dimension_semantics=("parallel",),
            vmem_limit_bytes=64 * 1024 * 1024,
        ),
    )(x2d, w2d, b2d)
    return out.reshape(*lead, c)


# speedup vs baseline: 1.2998x; 1.2998x over previous
from functools import partial

import jax
import jax.numpy as jnp
from jax.experimental import pallas as pl
from jax.experimental.pallas import tpu as pltpu

_BM = 6272
_NB = 4
_NCH = 2
_VMEM_LIMIT = 100 * 1024 * 1024


def _ln_slab(x, w, b, inv_c, eps):
    s1 = jnp.sum(x, axis=-1, keepdims=True)
    s2 = jnp.sum(x * x, axis=-1, keepdims=True)
    mean = s1 * inv_c
    var = s2 * inv_c - mean * mean
    rstd = jax.lax.rsqrt(jnp.maximum(var, 0.0) + eps)
    return (x - mean) * rstd * w + b


def _ln_stream_kernel(x_hbm, w_ref, b_ref, o_hbm,
                      in_buf, out_buf, in_sem, out_sem,
                      *, rows_half, inv_c, eps):
    core = pl.program_id(0)
    base = core * rows_half
    ch_rows = _BM // _NCH

    def in_cp(k, slot):
        return pltpu.make_async_copy(
            x_hbm.at[pl.ds(base + k * _BM, _BM), :],
            in_buf.at[slot],
            in_sem.at[slot])

    def out_cp(k, slot, ch):
        return pltpu.make_async_copy(
            out_buf.at[slot, pl.ds(ch * ch_rows, ch_rows), :],
            o_hbm.at[pl.ds(base + k * _BM + ch * ch_rows, ch_rows), :],
            out_sem.at[slot, ch])

    in_cp(0, 0).start()
    in_cp(1, 1).start()
    w = w_ref[...]
    b = b_ref[...]
    for k in range(_NB):
        slot = k % 2
        in_cp(k, slot).wait()
        for ch in range(_NCH):
            x = in_buf[slot, ch * ch_rows:(ch + 1) * ch_rows, :]
            y = _ln_slab(x, w, b, inv_c, eps)
            if k >= 2:
                out_cp(k - 2, slot, ch).wait()
            out_buf[slot, ch * ch_rows:(ch + 1) * ch_rows, :] = y
            out_cp(k, slot, ch).start()
        if k + 2 < _NB:
            in_cp(k + 2, slot).start()
    for k in (_NB - 2, _NB - 1):
        for ch in range(_NCH):
            out_cp(k, k % 2, ch).wait()


def _ln_block_kernel(x_ref, w_ref, b_ref, o_ref, *, inv_c, eps):
    o_ref[...] = _ln_slab(x_ref[...], w_ref[...], b_ref[...], inv_c, eps)


def kernel(x, weight, bias):
    eps = 1e-6
    c = x.shape[-1]
    lead = x.shape[:-1]
    rows = 1
    for d in lead:
        rows *= d
    x2d = x.reshape(rows, c)
    w2d = weight.reshape(1, c).astype(jnp.float32)
    b2d = bias.reshape(1, c).astype(jnp.float32)

    if rows == 2 * _NB * _BM:
        rows_half = rows // 2
        out = pl.pallas_call(
            partial(_ln_stream_kernel, rows_half=rows_half,
                    inv_c=1.0 / c, eps=eps),
            out_shape=jax.ShapeDtypeStruct((rows, c), x.dtype),
            grid=(2,),
            in_specs=[
                pl.BlockSpec(memory_space=pl.ANY),
                pl.BlockSpec((1, c), lambda i: (0, 0)),
                pl.BlockSpec((1, c), lambda i: (0, 0)),
            ],
            out_specs=pl.BlockSpec(memory_space=pl.ANY),
            scratch_shapes=[
                pltpu.VMEM((2, _BM, c), jnp.float32),
                pltpu.VMEM((2, _BM, c), jnp.float32),
                pltpu.SemaphoreType.DMA((2,)),
                pltpu.SemaphoreType.DMA((2, _NCH)),
            ],
            compiler_params=pltpu.CompilerParams(
                dimension_semantics=("parallel",),
                vmem_limit_bytes=_VMEM_LIMIT,
            ),
        )(x2d, w2d, b2d)
        return out.reshape(*lead, c)

    tm = max(8, min(rows, 2048))
    out = pl.pallas_call(
        partial(_ln_block_kernel, inv_c=1.0 / c, eps=eps),
        out_shape=jax.ShapeDtypeStruct((rows, c), x.dtype),
        grid=(pl.cdiv(rows, tm),),
        in_specs=[
            pl.BlockSpec((tm, c), lambda i: (i, 0)),
            pl.BlockSpec((1, c), lambda i: (0, 0)),
            pl.BlockSpec((1, c), lambda i: (0, 0)),
        ],
        out_specs=pl.BlockSpec((tm, c), lambda i: (i, 0)),
        compiler_params=pltpu.CompilerParams(
            dimension_semantics=("parallel",),
            vmem_limit_bytes=64 * 1024 * 1024,
        ),
    )(x2d, w2d, b2d)
    return out.reshape(*lead, c)


# speedup vs baseline: 1.3834x; 1.0643x over previous
from functools import partial

import jax
import jax.numpy as jnp
from jax.experimental import pallas as pl
from jax.experimental.pallas import tpu as pltpu

_TILE = 3136
_NT = 8
_DEPTH = 3
_VMEM_LIMIT = 100 * 1024 * 1024


def _ln_slab(x, w, b, inv_c, eps):
    s1 = jnp.sum(x, axis=-1, keepdims=True)
    s2 = jnp.sum(x * x, axis=-1, keepdims=True)
    mean = s1 * inv_c
    var = s2 * inv_c - mean * mean
    rstd = jax.lax.rsqrt(jnp.maximum(var, 0.0) + eps)
    return (x - mean) * rstd * w + b


def _ln_stream_kernel(x_hbm, w_ref, b_ref, o_hbm,
                      in_buf, out_buf, in_sem, out_sem,
                      *, rows_half, inv_c, eps):
    core = pl.program_id(0)
    base = core * rows_half

    def in_cp(t):
        slot = t % _DEPTH
        return pltpu.make_async_copy(
            x_hbm.at[pl.ds(base + t * _TILE, _TILE), :],
            in_buf.at[slot],
            in_sem.at[slot])

    def out_cp(t):
        slot = t % _DEPTH
        return pltpu.make_async_copy(
            out_buf.at[slot],
            o_hbm.at[pl.ds(base + t * _TILE, _TILE), :],
            out_sem.at[slot])

    for t in range(_DEPTH - 1):
        in_cp(t).start()
    w = w_ref[...]
    b = b_ref[...]
    for t in range(_NT):
        if t + _DEPTH - 1 < _NT:
            in_cp(t + _DEPTH - 1).start()
        in_cp(t).wait()
        y = _ln_slab(in_buf[t % _DEPTH], w, b, inv_c, eps)
        if t >= _DEPTH:
            out_cp(t - _DEPTH).wait()
        out_buf[t % _DEPTH] = y
        out_cp(t).start()
    for t in range(max(0, _NT - _DEPTH), _NT):
        out_cp(t).wait()


def _ln_block_kernel(x_ref, w_ref, b_ref, o_ref, *, inv_c, eps):
    o_ref[...] = _ln_slab(x_ref[...], w_ref[...], b_ref[...], inv_c, eps)


def kernel(x, weight, bias):
    eps = 1e-6
    c = x.shape[-1]
    lead = x.shape[:-1]
    rows = 1
    for d in lead:
        rows *= d
    x2d = x.reshape(rows, c)
    w2d = weight.reshape(1, c).astype(jnp.float32)
    b2d = bias.reshape(1, c).astype(jnp.float32)

    if rows == 2 * _NT * _TILE:
        rows_half = rows // 2
        out = pl.pallas_call(
            partial(_ln_stream_kernel, rows_half=rows_half,
                    inv_c=1.0 / c, eps=eps),
            out_shape=jax.ShapeDtypeStruct((rows, c), x.dtype),
            grid=(2,),
            in_specs=[
                pl.BlockSpec(memory_space=pl.ANY),
                pl.BlockSpec((1, c), lambda i: (0, 0)),
                pl.BlockSpec((1, c), lambda i: (0, 0)),
            ],
            out_specs=pl.BlockSpec(memory_space=pl.ANY),
            scratch_shapes=[
                pltpu.VMEM((_DEPTH, _TILE, c), jnp.float32),
                pltpu.VMEM((_DEPTH, _TILE, c), jnp.float32),
                pltpu.SemaphoreType.DMA((_DEPTH,)),
                pltpu.SemaphoreType.DMA((_DEPTH,)),
            ],
            compiler_params=pltpu.CompilerParams(
                dimension_semantics=("parallel",),
                vmem_limit_bytes=_VMEM_LIMIT,
            ),
        )(x2d, w2d, b2d)
        return out.reshape(*lead, c)

    tm = max(8, min(rows, 2048))
    out = pl.pallas_call(
        partial(_ln_block_kernel, inv_c=1.0 / c, eps=eps),
        out_shape=jax.ShapeDtypeStruct((rows, c), x.dtype),
        grid=(pl.cdiv(rows, tm),),
        in_specs=[
            pl.BlockSpec((tm, c), lambda i: (i, 0)),
            pl.BlockSpec((1, c), lambda i: (0, 0)),
            pl.BlockSpec((1, c), lambda i: (0, 0)),
        ],
        out_specs=pl.BlockSpec((tm, c), lambda i: (i, 0)),
        compiler_params=pltpu.CompilerParams(
            dimension_semantics=("parallel",),
            vmem_limit_bytes=64 * 1024 * 1024,
        ),
    )(x2d, w2d, b2d)
    return out.reshape(*lead, c)
